# Initial kernel scaffold; baseline (speedup 1.0000x reference)
import jax
import jax.numpy as jnp
from jax.experimental import pallas as pl


def kernel(x, weight, e_score_correction_bias):
    raise NotImplementedError("write your pallas kernel here")


# speedup vs baseline: 1.0912x; 1.0912x over previous
import jax
import jax.numpy as jnp
from jax.experimental import pallas as pl
from jax.experimental.pallas import tpu as pltpu

_N_EXPERTS = 64
_TOP_K = 8
_N_GROUP = 8
_GROUP_SIZE = _N_EXPERTS // _N_GROUP
_N_DROP_GROUP = 4
_SCALE = 2.5
_BLOCK = 256


def _router_body(x_ref, w_ref, b_ref, inds_ref, sel_ref):
    blk = x_ref.shape[0]
    gates = jax.lax.dot_general(
        x_ref[...], w_ref[...], (((1,), (1,)), ((), ())),
        preferred_element_type=jnp.float32,
    )
    orig = jax.nn.sigmoid(gates)
    scores = orig + b_ref[...]

    iota_g = jax.lax.broadcasted_iota(jnp.int32, (blk, _GROUP_SIZE), 1)
    iota_grp = jax.lax.broadcasted_iota(jnp.int32, (blk, _N_GROUP), 1)
    iota_e = jax.lax.broadcasted_iota(jnp.int32, (blk, _N_EXPERTS), 1)

    gs_cols = []
    for g in range(_N_GROUP):
        sl = scores[:, g * _GROUP_SIZE:(g + 1) * _GROUP_SIZE]
        m1 = jnp.max(sl, axis=1, keepdims=True)
        fidx = jnp.min(
            jnp.where(sl == m1, iota_g, _GROUP_SIZE), axis=1, keepdims=True)
        m2 = jnp.max(
            jnp.where(iota_g == fidx, -jnp.inf, sl), axis=1, keepdims=True)
        gs_cols.append(m1 + m2)
    gs = jnp.concatenate(gs_cols, axis=1)

    rank = jnp.zeros((blk, _N_GROUP), jnp.int32)
    for h in range(_N_GROUP):
        gh = gs[:, h:h + 1]
        beats = (gh < gs) | ((gh == gs) & (iota_grp > h))
        rank = rank + jnp.where(beats, 1, 0)
    drop = rank < _N_DROP_GROUP

    keep = jnp.zeros((blk, _N_EXPERTS), jnp.bool_)
    for g in range(_N_GROUP):
        keep = keep | ((iota_e == g * _GROUP_SIZE) & drop[:, g:g + 1])
    scores = jnp.where(keep, scores, 0.0)

    work = scores
    idx_cols, val_cols = [], []
    for _ in range(_TOP_K):
        m = jnp.min(work, axis=1, keepdims=True)
        idx = jnp.min(
            jnp.where(work == m, iota_e, _N_EXPERTS), axis=1, keepdims=True)
        hit = iota_e == idx
        val = jnp.sum(jnp.where(hit, orig, 0.0), axis=1, keepdims=True)
        work = jnp.where(hit, jnp.inf, work)
        idx_cols.append(idx)
        val_cols.append(val)
    inds = jnp.concatenate(idx_cols, axis=1)
    sel = jnp.concatenate(val_cols, axis=1)
    denom = jnp.sum(sel, axis=1, keepdims=True)
    sel = sel / (denom + 1e-20) * _SCALE

    inds_ref[...] = inds
    sel_ref[...] = sel


def kernel(x, weight, e_score_correction_bias, interpret=False):
    n_tokens, d_model = x.shape
    n_experts = weight.shape[0]
    bias2d = e_score_correction_bias.reshape(1, n_experts)
    grid = (n_tokens // _BLOCK,)
    inds, sel = pl.pallas_call(
        _router_body,
        grid=grid,
        in_specs=[
            pl.BlockSpec((_BLOCK, d_model), lambda i: (i, 0)),
            pl.BlockSpec((n_experts, d_model), lambda i: (0, 0)),
            pl.BlockSpec((1, n_experts), lambda i: (0, 0)),
        ],
        out_specs=[
            pl.BlockSpec((_BLOCK, _TOP_K), lambda i: (i, 0)),
            pl.BlockSpec((_BLOCK, _TOP_K), lambda i: (i, 0)),
        ],
        out_shape=[
            jax.ShapeDtypeStruct((n_tokens, _TOP_K), jnp.int32),
            jax.ShapeDtypeStruct((n_tokens, _TOP_K), jnp.float32),
        ],
        compiler_params=pltpu.CompilerParams(
            dimension_semantics=("parallel",)),
        interpret=interpret,
    )(x, weight, bias2d)
    return inds, sel


# speedup vs baseline: 1.7763x; 1.6278x over previous
import numpy as np

import jax
import jax.numpy as jnp
from jax.experimental import pallas as pl
from jax.experimental.pallas import tpu as pltpu

_N_EXPERTS = 64
_TOP_K = 8
_N_GROUP = 8
_GROUP_SIZE = _N_EXPERTS // _N_GROUP
_N_DROP_GROUP = 4
_SCALE = 2.5
_BLOCK = 256


_CAND_IDS = np.array(
    [g * _GROUP_SIZE for g in range(_N_GROUP)] + [1, 2, 3, 4, 5, 6, 7, 9],
    dtype=np.int32)

def _xor_perm(k):
    p = np.zeros((_N_EXPERTS, _N_EXPERTS), np.float32)
    for j in range(_N_EXPERTS):
        p[j ^ k, j] = 1.0
    return p

_P1, _P2, _P4 = _xor_perm(1), _xor_perm(2), _xor_perm(4)
_S_HEAD = np.zeros((_N_EXPERTS, _N_GROUP), np.float32)
for _g in range(_N_GROUP):
    _S_HEAD[_g * _GROUP_SIZE, _g] = 1.0
_S_CAND = np.zeros((_N_EXPERTS, 16), np.float32)
for _c, _e in enumerate(_CAND_IDS):
    _S_CAND[_e, _c] = 1.0


def _router_body(x_ref, w_ref, b_ref, p1_ref, p2_ref, p4_ref,
                 shead_ref, scand_ref, ids_ref, inds_ref, sel_ref):
    blk = x_ref.shape[0]
    f32 = jnp.float32
    gates = jax.lax.dot_general(
        x_ref[...], w_ref[...], (((1,), (1,)), ((), ())),
        preferred_element_type=f32,
    )
    orig = jax.nn.sigmoid(gates)
    scores = orig + b_ref[...]

    def mm(a, m_ref):
        return jax.lax.dot_general(
            a, m_ref[...], (((1,), (0,)), ((), ())),
            preferred_element_type=f32,
            precision=jax.lax.Precision.HIGHEST)

    m1 = scores
    m2 = jnp.full_like(scores, -1e30)
    for p in (p1_ref, p2_ref, p4_ref):
        m1p = mm(m1, p)
        m2p = mm(m2, p)
        m2 = jnp.maximum(jnp.minimum(m1, m1p), jnp.maximum(m2, m2p))
        m1 = jnp.maximum(m1, m1p)
    gs = mm(m1 + m2, shead_ref)

    iota_grp = jax.lax.broadcasted_iota(jnp.int32, (blk, _N_GROUP), 1)
    grank = jnp.zeros((blk, _N_GROUP), jnp.int32)
    for h in range(_N_GROUP):
        gh = gs[:, h:h + 1]
        beats = (gh < gs) | ((gh == gs) & (iota_grp > h))
        grank = grank + jnp.where(beats, 1, 0)
    drop = grank < _N_DROP_GROUP

    heads = mm(scores, shead_ref)
    heads_masked = jnp.where(drop, heads, 0.0)
    cand = jnp.concatenate(
        [heads_masked, jnp.zeros((blk, 8), f32)], axis=1)
    orig_cand = mm(orig, scand_ref)

    ids_row = ids_ref[...]
    b = jax.lax.bitcast_convert_type(cand, jnp.int32)
    mono = jnp.where(b < 0, jnp.int32(-2147483648) - b, b)
    key = (mono & jnp.int32(-64)) | ids_row

    crank = jnp.zeros((blk, 16), jnp.int32)
    for c in range(16):
        kc = key[:, c:c + 1]
        crank = crank + jnp.where(kc < key, 1, 0)

    ind_cols, val_cols = [], []
    for r in range(_TOP_K):
        mr = crank == r
        ind_cols.append(jnp.sum(
            jnp.where(mr, ids_row, 0), axis=1, keepdims=True))
        val_cols.append(jnp.sum(
            jnp.where(mr, orig_cand, 0.0), axis=1, keepdims=True))
    inds = jnp.concatenate(ind_cols, axis=1)
    sel = jnp.concatenate(val_cols, axis=1)
    denom = jnp.sum(sel, axis=1, keepdims=True)
    sel = sel / (denom + 1e-20) * _SCALE

    inds_ref[...] = inds
    sel_ref[...] = sel


def kernel(x, weight, e_score_correction_bias, interpret=False):
    n_tokens, d_model = x.shape
    n_experts = weight.shape[0]
    bias2d = e_score_correction_bias.reshape(1, n_experts)
    grid = (n_tokens // _BLOCK,)
    consts = (jnp.asarray(_P1), jnp.asarray(_P2), jnp.asarray(_P4),
              jnp.asarray(_S_HEAD), jnp.asarray(_S_CAND),
              jnp.asarray(_CAND_IDS.reshape(1, 16)))
    const_specs = [
        pl.BlockSpec(c.shape, lambda i: (0, 0)) for c in consts]
    inds, sel = pl.pallas_call(
        _router_body,
        grid=grid,
        in_specs=[
            pl.BlockSpec((_BLOCK, d_model), lambda i: (i, 0)),
            pl.BlockSpec((n_experts, d_model), lambda i: (0, 0)),
            pl.BlockSpec((1, n_experts), lambda i: (0, 0)),
        ] + const_specs,
        out_specs=[
            pl.BlockSpec((_BLOCK, _TOP_K), lambda i: (i, 0)),
            pl.BlockSpec((_BLOCK, _TOP_K), lambda i: (i, 0)),
        ],
        out_shape=[
            jax.ShapeDtypeStruct((n_tokens, _TOP_K), jnp.int32),
            jax.ShapeDtypeStruct((n_tokens, _TOP_K), jnp.float32),
        ],
        compiler_params=pltpu.CompilerParams(
            dimension_semantics=("parallel",)),
        interpret=interpret,
    )(x, weight, bias2d, *consts)
    return inds, sel


# speedup vs baseline: 2.0445x; 1.1510x over previous
import numpy as np

import jax
import jax.numpy as jnp
from jax.experimental import pallas as pl
from jax.experimental.pallas import tpu as pltpu

_N_EXPERTS = 64
_TOP_K = 8
_N_GROUP = 8
_GROUP_SIZE = _N_EXPERTS // _N_GROUP
_N_DROP_GROUP = 4
_SCALE = 2.5
_BLOCK = 256

_HEAD_IDS = [g * _GROUP_SIZE for g in range(_N_GROUP)]
_ZERO_IDS = [1, 2, 3, 4, 5, 6, 7, 9]
_CAND_IDS = np.array(_HEAD_IDS + _ZERO_IDS, dtype=np.float32)

_S_HEAD = np.zeros((_N_EXPERTS, _N_GROUP), np.float32)
for _g in range(_N_GROUP):
    _S_HEAD[_g * _GROUP_SIZE, _g] = 1.0
_S_CAND = np.zeros((_N_EXPERTS, 16), np.float32)
for _c, _e in enumerate(np.array(_HEAD_IDS + _ZERO_IDS)):
    _S_CAND[_e, _c] = 1.0
_R8 = np.zeros((8, 64), np.float32)
_M64 = np.zeros((64, 8), np.float32)
for _i in range(64):
    _R8[_i // 8, _i] = 1.0
    _M64[_i, _i % 8] = 1.0
_BD_IDS = np.zeros((128, 8), np.float32)
_BD_ONES9 = np.zeros((128, 9), np.float32)
for _r in range(8):
    for _c in range(16):
        _BD_IDS[_r * 16 + _c, _r] = float(_CAND_IDS[_c])
        _BD_ONES9[_r * 16 + _c, _r] = 1.0
_BD_ONES9[:, 8] = 1.0
_KZ = np.array([sum(1 for z in _ZERO_IDS if z < h) for h in _HEAD_IDS],
               np.float32).reshape(1, 8)

_ONES8 = np.ones((8, 8), np.float32)

_CONSTS = (_S_HEAD, _S_CAND, _R8, _M64, _BD_IDS, _BD_ONES9, _KZ, _ONES8)


def _router_body(x_ref, w_ref, b_ref, shead_ref, scand_ref, r8_ref,
                 m64_ref, bdids_ref, bdones_ref, kz_ref, ones8_ref,
                 inds_ref, sel_ref):
    blk = x_ref.shape[0]
    f32 = jnp.float32
    gates = jax.lax.dot_general(
        x_ref[...], w_ref[...], (((1,), (1,)), ((), ())),
        preferred_element_type=f32,
    )
    orig = jax.nn.sigmoid(gates)
    scores = orig + b_ref[...]

    def mm_h(a, m_ref):
        return jax.lax.dot_general(
            a, m_ref[...], (((1,), (0,)), ((), ())),
            preferred_element_type=f32,
            precision=jax.lax.Precision.HIGHEST)

    def mm_d(a, m_ref):
        return jax.lax.dot_general(
            a, m_ref[...], (((1,), (0,)), ((), ())),
            preferred_element_type=f32)

    m1 = scores
    m2 = jnp.full_like(scores, -1e30)
    for s in (1, 2, 4):
        m1p = pltpu.roll(m1, _N_EXPERTS - s, 1)
        m2p = pltpu.roll(m2, _N_EXPERTS - s, 1)
        m2 = jnp.maximum(jnp.minimum(m1, m1p), jnp.maximum(m2, m2p))
        m1 = jnp.maximum(m1, m1p)
    gs = mm_h(m1 + m2, shead_ref)

    iota64 = jax.lax.broadcasted_iota(jnp.int32, (1, 64), 1)
    tie_pat = (iota64 // 8) < (iota64 % 8)

    def asc_rank8(v):
        a = jnp.concatenate([v] * 8, axis=1)
        b = mm_h(v, r8_ref)
        beats = (b < a) | ((b == a) & tie_pat)
        return mm_d(jnp.where(beats, 1.0, 0.0), m64_ref)

    drop = asc_rank8(gs) < (_N_DROP_GROUP - 0.5)

    heads = jnp.where(drop, mm_h(scores, shead_ref), 0.0)

    hh_rank = asc_rank8(heads)
    zbelow = jnp.where(heads > 0.0, 8.0,
                       jnp.where(heads == 0.0, kz_ref[...], 0.0))
    head_rank = hh_rank + zbelow

    iota8 = jax.lax.broadcasted_iota(jnp.int32, (1, 8), 1)
    nneg = mm_d(jnp.where(heads < 0.0, 1.0, 0.0), ones8_ref)
    e0 = jnp.where(heads[:, 0:1] == 0.0, 1.0, 0.0)
    e1 = jnp.where((heads[:, 1:2] == 0.0) & (iota8 == 7), 1.0, 0.0)
    zero_rank = nneg + e0 + e1 + iota8.astype(f32)

    crank = jnp.concatenate([head_rank, zero_rank], axis=1)
    orig_cand = mm_h(orig, scand_ref)

    crank_rep = jnp.concatenate([crank] * 8, axis=1)
    orig_rep = jnp.concatenate([orig_cand] * 8, axis=1)
    iota128 = jax.lax.broadcasted_iota(jnp.int32, (1, 128), 1)
    seg = (iota128 // 16).astype(f32)
    hit = crank_rep == seg
    inds = mm_d(jnp.where(hit, 1.0, 0.0), bdids_ref).astype(jnp.int32)
    valsden = mm_h(jnp.where(hit, orig_rep, 0.0), bdones_ref)
    denom = valsden[:, 8:9]
    sel = valsden[:, 0:8] / (denom + 1e-20) * _SCALE

    inds_ref[...] = inds
    sel_ref[...] = sel


def kernel(x, weight, e_score_correction_bias, interpret=False):
    n_tokens, d_model = x.shape
    n_experts = weight.shape[0]
    bias2d = e_score_correction_bias.reshape(1, n_experts)
    grid = (n_tokens // _BLOCK,)
    consts = tuple(jnp.asarray(c) for c in _CONSTS)
    const_specs = [
        pl.BlockSpec(c.shape, lambda i: (0, 0)) for c in consts]
    inds, sel = pl.pallas_call(
        _router_body,
        grid=grid,
        in_specs=[
            pl.BlockSpec((_BLOCK, d_model), lambda i: (i, 0)),
            pl.BlockSpec((n_experts, d_model), lambda i: (0, 0)),
            pl.BlockSpec((1, n_experts), lambda i: (0, 0)),
        ] + const_specs,
        out_specs=[
            pl.BlockSpec((_BLOCK, _TOP_K), lambda i: (i, 0)),
            pl.BlockSpec((_BLOCK, _TOP_K), lambda i: (i, 0)),
        ],
        out_shape=[
            jax.ShapeDtypeStruct((n_tokens, _TOP_K), jnp.int32),
            jax.ShapeDtypeStruct((n_tokens, _TOP_K), jnp.float32),
        ],
        compiler_params=pltpu.CompilerParams(
            dimension_semantics=("parallel",)),
        interpret=interpret,
    )(x, weight, bias2d, *consts)
    return inds, sel


# speedup vs baseline: 2.4829x; 1.2145x over previous
import numpy as np

import jax
import jax.numpy as jnp
from jax.experimental import pallas as pl
from jax.experimental.pallas import tpu as pltpu

_N_EXPERTS = 64
_TOP_K = 8
_N_GROUP = 8
_GROUP_SIZE = _N_EXPERTS // _N_GROUP
_N_DROP_GROUP = 4
_SCALE = 2.5
_BLOCK = 512

_HEAD_IDS = [g * _GROUP_SIZE for g in range(_N_GROUP)]
_ZERO_IDS = [1, 2, 3, 4, 5, 6, 7, 9]
_CAND_IDS = np.array(_HEAD_IDS + _ZERO_IDS, dtype=np.float32)

_S_HEAD = np.zeros((_N_EXPERTS, _N_GROUP), np.float32)
for _g in range(_N_GROUP):
    _S_HEAD[_g * _GROUP_SIZE, _g] = 1.0
_S_GH = np.zeros((2 * _N_EXPERTS, 2 * _N_GROUP), np.float32)
_S_GH[:_N_EXPERTS, :_N_GROUP] = _S_HEAD
_S_GH[_N_EXPERTS:, _N_GROUP:] = _S_HEAD
_S_CAND = np.zeros((_N_EXPERTS, 16), np.float32)
for _c, _e in enumerate(np.array(_HEAD_IDS + _ZERO_IDS)):
    _S_CAND[_e, _c] = 1.0
_R8 = np.zeros((8, 64), np.float32)
_M64 = np.zeros((64, 8), np.float32)
for _i in range(64):
    _R8[_i // 8, _i] = 1.0
    _M64[_i, _i % 8] = 1.0
_BD_IDS = np.zeros((128, 8), np.float32)
_BD_ONES9 = np.zeros((128, 9), np.float32)
for _r in range(8):
    for _c in range(16):
        _BD_IDS[_r * 16 + _c, _r] = float(_CAND_IDS[_c])
        _BD_ONES9[_r * 16 + _c, _r] = 1.0
_BD_ONES9[:, 8] = 1.0
_KZ = np.array([sum(1 for z in _ZERO_IDS if z < h) for h in _HEAD_IDS],
               np.float32).reshape(1, 8)

_ONES8 = np.ones((8, 8), np.float32)

_CONSTS = (_S_GH, _S_CAND, _R8, _M64, _BD_IDS, _BD_ONES9, _KZ, _ONES8)


def _router_body(x_ref, w_ref, b_ref, sgh_ref, scand_ref, r8_ref,
                 m64_ref, bdids_ref, bdones_ref, kz_ref, ones8_ref,
                 inds_ref, sel_ref):
    blk = x_ref.shape[0]
    f32 = jnp.float32
    gates = jax.lax.dot_general(
        x_ref[...], w_ref[...], (((1,), (1,)), ((), ())),
        preferred_element_type=f32,
    )
    orig = jax.nn.sigmoid(gates)
    scores = orig + b_ref[...]

    def mm_h(a, m_ref):
        return jax.lax.dot_general(
            a, m_ref[...], (((1,), (0,)), ((), ())),
            preferred_element_type=f32,
            precision=jax.lax.Precision.HIGHEST)

    def mm_d(a, m_ref):
        return jax.lax.dot_general(
            a, m_ref[...], (((1,), (0,)), ((), ())),
            preferred_element_type=f32)

    m1p = pltpu.roll(scores, _N_EXPERTS - 1, 1)
    m1 = jnp.maximum(scores, m1p)
    m2 = jnp.minimum(scores, m1p)
    for s in (2, 4):
        m1p = pltpu.roll(m1, _N_EXPERTS - s, 1)
        m2p = pltpu.roll(m2, _N_EXPERTS - s, 1)
        m2 = jnp.maximum(jnp.minimum(m1, m1p), jnp.maximum(m2, m2p))
        m1 = jnp.maximum(m1, m1p)
    gh = mm_h(jnp.concatenate([m1 + m2, scores], axis=1), sgh_ref)
    gs = gh[:, 0:_N_GROUP]

    iota64 = jax.lax.broadcasted_iota(jnp.int32, (1, 64), 1)
    tie_pat = (iota64 // 8) < (iota64 % 8)

    def asc_rank8(v):
        a = jnp.concatenate([v] * 8, axis=1)
        b = mm_h(v, r8_ref)
        beats = (b < a) | ((b == a) & tie_pat)
        return mm_d(jnp.where(beats, 1.0, 0.0), m64_ref)

    drop = asc_rank8(gs) < (_N_DROP_GROUP - 0.5)

    heads = jnp.where(drop, gh[:, _N_GROUP:], 0.0)

    hh_rank = asc_rank8(heads)
    zbelow = jnp.where(heads > 0.0, 8.0,
                       jnp.where(heads == 0.0, kz_ref[...], 0.0))
    head_rank = hh_rank + zbelow

    iota8 = jax.lax.broadcasted_iota(jnp.int32, (1, 8), 1)
    nneg = mm_d(jnp.where(heads < 0.0, 1.0, 0.0), ones8_ref)
    e0 = jnp.where(heads[:, 0:1] == 0.0, 1.0, 0.0)
    e1 = jnp.where((heads[:, 1:2] == 0.0) & (iota8 == 7), 1.0, 0.0)
    zero_rank = nneg + e0 + e1 + iota8.astype(f32)

    crank = jnp.concatenate([head_rank, zero_rank], axis=1)
    orig_cand = mm_h(orig, scand_ref)

    crank_rep = jnp.concatenate([crank] * 8, axis=1)
    orig_rep = jnp.concatenate([orig_cand] * 8, axis=1)
    iota128 = jax.lax.broadcasted_iota(jnp.int32, (1, 128), 1)
    seg = (iota128 // 16).astype(f32)
    hit = crank_rep == seg
    inds = mm_d(jnp.where(hit, 1.0, 0.0), bdids_ref).astype(jnp.int32)
    valsden = mm_h(jnp.where(hit, orig_rep, 0.0), bdones_ref)
    denom = valsden[:, 8:9]
    sel = valsden[:, 0:8] / (denom + 1e-20) * _SCALE

    inds_ref[...] = inds
    sel_ref[...] = sel


def kernel(x, weight, e_score_correction_bias, interpret=False):
    n_tokens, d_model = x.shape
    n_experts = weight.shape[0]
    bias2d = e_score_correction_bias.reshape(1, n_experts)
    grid = (n_tokens // _BLOCK,)
    consts = tuple(jnp.asarray(c) for c in _CONSTS)
    const_specs = [
        pl.BlockSpec(c.shape, lambda i: (0, 0)) for c in consts]
    inds, sel = pl.pallas_call(
        _router_body,
        grid=grid,
        in_specs=[
            pl.BlockSpec((_BLOCK, d_model), lambda i: (i, 0)),
            pl.BlockSpec((n_experts, d_model), lambda i: (0, 0)),
            pl.BlockSpec((1, n_experts), lambda i: (0, 0)),
        ] + const_specs,
        out_specs=[
            pl.BlockSpec((_BLOCK, _TOP_K), lambda i: (i, 0)),
            pl.BlockSpec((_BLOCK, _TOP_K), lambda i: (i, 0)),
        ],
        out_shape=[
            jax.ShapeDtypeStruct((n_tokens, _TOP_K), jnp.int32),
            jax.ShapeDtypeStruct((n_tokens, _TOP_K), jnp.float32),
        ],
        compiler_params=pltpu.CompilerParams(
            dimension_semantics=("parallel",)),
        interpret=interpret,
    )(x, weight, bias2d, *consts)
    return inds, sel


# speedup vs baseline: 2.5899x; 1.0431x over previous
import numpy as np

import jax
import jax.numpy as jnp
from jax.experimental import pallas as pl
from jax.experimental.pallas import tpu as pltpu

_N_EXPERTS = 64
_TOP_K = 8
_N_GROUP = 8
_GROUP_SIZE = _N_EXPERTS // _N_GROUP
_N_DROP_GROUP = 4
_SCALE = 2.5
_BLOCK = 1024

_HEAD_IDS = [g * _GROUP_SIZE for g in range(_N_GROUP)]
_ZERO_IDS = [1, 2, 3, 4, 5, 6, 7, 9]
_CAND_IDS = np.array(_HEAD_IDS + _ZERO_IDS, dtype=np.float32)

_S_HEAD = np.zeros((_N_EXPERTS, _N_GROUP), np.float32)
for _g in range(_N_GROUP):
    _S_HEAD[_g * _GROUP_SIZE, _g] = 1.0
_S_GH = np.zeros((2 * _N_EXPERTS, 2 * _N_GROUP), np.float32)
_S_GH[:_N_EXPERTS, :_N_GROUP] = _S_HEAD
_S_GH[_N_EXPERTS:, _N_GROUP:] = _S_HEAD
_S_CAND = np.zeros((_N_EXPERTS, 16), np.float32)
for _c, _e in enumerate(np.array(_HEAD_IDS + _ZERO_IDS)):
    _S_CAND[_e, _c] = 1.0
_R8 = np.zeros((8, 64), np.float32)
_M64 = np.zeros((64, 8), np.float32)
for _i in range(64):
    _R8[_i // 8, _i] = 1.0
    _M64[_i, _i % 8] = 1.0
_BD_IDS = np.zeros((128, 8), np.float32)
_BD_ONES9 = np.zeros((128, 9), np.float32)
for _r in range(8):
    for _c in range(16):
        _BD_IDS[_r * 16 + _c, _r] = float(_CAND_IDS[_c])
        _BD_ONES9[_r * 16 + _c, _r] = 1.0
_BD_ONES9[:, 8] = 1.0
_KZ = np.array([sum(1 for z in _ZERO_IDS if z < h) for h in _HEAD_IDS],
               np.float32).reshape(1, 8)

_ONES8 = np.ones((8, 8), np.float32)

_CONSTS = (_S_GH, _S_CAND, _R8, _M64, _BD_IDS, _BD_ONES9, _KZ, _ONES8)


def _router_body(x_ref, w_ref, b_ref, sgh_ref, scand_ref, r8_ref,
                 m64_ref, bdids_ref, bdones_ref, kz_ref, ones8_ref,
                 inds_ref, sel_ref):
    blk = x_ref.shape[0]
    f32 = jnp.float32
    gates = jax.lax.dot_general(
        x_ref[...], w_ref[...], (((1,), (1,)), ((), ())),
        preferred_element_type=f32,
    )
    orig = jax.nn.sigmoid(gates)
    scores = orig + b_ref[...]

    def mm_h(a, m_ref):
        return jax.lax.dot_general(
            a, m_ref[...], (((1,), (0,)), ((), ())),
            preferred_element_type=f32,
            precision=jax.lax.Precision.HIGHEST)

    def mm_d(a, m_ref):
        return jax.lax.dot_general(
            a, m_ref[...], (((1,), (0,)), ((), ())),
            preferred_element_type=f32)

    m1p = pltpu.roll(scores, _N_EXPERTS - 1, 1)
    m1 = jnp.maximum(scores, m1p)
    m2 = jnp.minimum(scores, m1p)
    for s in (2, 4):
        m1p = pltpu.roll(m1, _N_EXPERTS - s, 1)
        m2p = pltpu.roll(m2, _N_EXPERTS - s, 1)
        m2 = jnp.maximum(jnp.minimum(m1, m1p), jnp.maximum(m2, m2p))
        m1 = jnp.maximum(m1, m1p)
    gh = mm_h(jnp.concatenate([m1 + m2, scores], axis=1), sgh_ref)
    gs = gh[:, 0:_N_GROUP]

    iota64 = jax.lax.broadcasted_iota(jnp.int32, (1, 64), 1)
    tie_pat = (iota64 // 8) < (iota64 % 8)

    def asc_rank8(v):
        a = jnp.concatenate([v] * 8, axis=1)
        b = mm_h(v, r8_ref)
        beats = (b < a) | ((b == a) & tie_pat)
        return mm_d(jnp.where(beats, 1.0, 0.0), m64_ref)

    drop = asc_rank8(gs) < (_N_DROP_GROUP - 0.5)

    heads = jnp.where(drop, gh[:, _N_GROUP:], 0.0)

    hh_rank = asc_rank8(heads)
    zbelow = jnp.where(heads > 0.0, 8.0,
                       jnp.where(heads == 0.0, kz_ref[...], 0.0))
    head_rank = hh_rank + zbelow

    iota8 = jax.lax.broadcasted_iota(jnp.int32, (1, 8), 1)
    nneg = mm_d(jnp.where(heads < 0.0, 1.0, 0.0), ones8_ref)
    e0 = jnp.where(heads[:, 0:1] == 0.0, 1.0, 0.0)
    e1 = jnp.where((heads[:, 1:2] == 0.0) & (iota8 == 7), 1.0, 0.0)
    zero_rank = nneg + e0 + e1 + iota8.astype(f32)

    crank = jnp.concatenate([head_rank, zero_rank], axis=1)
    orig_cand = mm_h(orig, scand_ref)

    crank_rep = jnp.concatenate([crank] * 8, axis=1)
    orig_rep = jnp.concatenate([orig_cand] * 8, axis=1)
    iota128 = jax.lax.broadcasted_iota(jnp.int32, (1, 128), 1)
    seg = (iota128 // 16).astype(f32)
    hit = crank_rep == seg
    inds = mm_d(jnp.where(hit, 1.0, 0.0), bdids_ref).astype(jnp.int32)
    valsden = mm_h(jnp.where(hit, orig_rep, 0.0), bdones_ref)
    denom = valsden[:, 8:9]
    sel = valsden[:, 0:8] / (denom + 1e-20) * _SCALE

    inds_ref[...] = inds
    sel_ref[...] = sel


def kernel(x, weight, e_score_correction_bias, interpret=False):
    n_tokens, d_model = x.shape
    n_experts = weight.shape[0]
    bias2d = e_score_correction_bias.reshape(1, n_experts)
    grid = (n_tokens // _BLOCK,)
    consts = tuple(jnp.asarray(c) for c in _CONSTS)
    const_specs = [
        pl.BlockSpec(c.shape, lambda i: (0, 0)) for c in consts]
    inds, sel = pl.pallas_call(
        _router_body,
        grid=grid,
        in_specs=[
            pl.BlockSpec((_BLOCK, d_model), lambda i: (i, 0)),
            pl.BlockSpec((n_experts, d_model), lambda i: (0, 0)),
            pl.BlockSpec((1, n_experts), lambda i: (0, 0)),
        ] + const_specs,
        out_specs=[
            pl.BlockSpec((_BLOCK, _TOP_K), lambda i: (i, 0)),
            pl.BlockSpec((_BLOCK, _TOP_K), lambda i: (i, 0)),
        ],
        out_shape=[
            jax.ShapeDtypeStruct((n_tokens, _TOP_K), jnp.int32),
            jax.ShapeDtypeStruct((n_tokens, _TOP_K), jnp.float32),
        ],
        compiler_params=pltpu.CompilerParams(
            dimension_semantics=("parallel",)),
        interpret=interpret,
    )(x, weight, bias2d, *consts)
    return inds, sel


# speedup vs baseline: 2.5919x; 1.0008x over previous
import numpy as np

import jax
import jax.numpy as jnp
from jax.experimental import pallas as pl
from jax.experimental.pallas import tpu as pltpu

_N_EXPERTS = 64
_TOP_K = 8
_N_GROUP = 8
_GROUP_SIZE = _N_EXPERTS // _N_GROUP
_N_DROP_GROUP = 4
_SCALE = 2.5
_BLOCK = 1024

_HEAD_IDS = [g * _GROUP_SIZE for g in range(_N_GROUP)]
_ZERO_IDS = [1, 2, 3, 4, 5, 6, 7, 9]
_CAND_IDS = np.array(_HEAD_IDS + _ZERO_IDS, dtype=np.float32)

_S_HEAD = np.zeros((_N_EXPERTS, _N_GROUP), np.float32)
for _g in range(_N_GROUP):
    _S_HEAD[_g * _GROUP_SIZE, _g] = 1.0
_S_GH = np.zeros((2 * _N_EXPERTS, 2 * _N_GROUP), np.float32)
for _g in range(_N_GROUP):
    _S_GH[_g * _GROUP_SIZE + _GROUP_SIZE - 1, _g] = 1.0
_S_GH[_N_EXPERTS:, _N_GROUP:] = _S_HEAD
_S_CAND = np.zeros((_N_EXPERTS, 16), np.float32)
for _c, _e in enumerate(np.array(_HEAD_IDS + _ZERO_IDS)):
    _S_CAND[_e, _c] = 1.0
_R8 = np.zeros((8, 64), np.float32)
_M64 = np.zeros((64, 8), np.float32)
for _i in range(64):
    _R8[_i // 8, _i] = 1.0
    _M64[_i, _i % 8] = 1.0
_BD_IDS = np.zeros((128, 8), np.float32)
_BD_ONES9 = np.zeros((128, 9), np.float32)
for _r in range(8):
    for _c in range(16):
        _BD_IDS[_r * 16 + _c, _r] = float(_CAND_IDS[_c])
        _BD_ONES9[_r * 16 + _c, _r] = 1.0
_BD_ONES9[:, 8] = 1.0
_KZ = np.array([sum(1 for z in _ZERO_IDS if z < h) for h in _HEAD_IDS],
               np.float32).reshape(1, 8)

_ONES8 = np.ones((8, 8), np.float32)

_CONSTS = (_S_GH, _S_CAND, _R8, _M64, _BD_IDS, _BD_ONES9, _KZ, _ONES8)


def _router_body(x_ref, w_ref, b_ref, sgh_ref, scand_ref, r8_ref,
                 m64_ref, bdids_ref, bdones_ref, kz_ref, ones8_ref,
                 inds_ref, sel_ref):
    blk = x_ref.shape[0]
    f32 = jnp.float32
    gates = jax.lax.dot_general(
        x_ref[...], w_ref[...], (((1,), (1,)), ((), ())),
        preferred_element_type=f32,
    )
    orig = jax.nn.sigmoid(gates)
    scores = orig + b_ref[...]

    def mm_h(a, m_ref):
        return jax.lax.dot_general(
            a, m_ref[...], (((1,), (0,)), ((), ())),
            preferred_element_type=f32,
            precision=jax.lax.Precision.HIGHEST)

    def mm_d(a, m_ref):
        return jax.lax.dot_general(
            a, m_ref[...], (((1,), (0,)), ((), ())),
            preferred_element_type=f32)

    m1p = pltpu.roll(scores, 1, 1)
    m1 = jnp.maximum(scores, m1p)
    m2 = jnp.minimum(scores, m1p)
    for s in (2, 4):
        m1p = pltpu.roll(m1, s, 1)
        m2p = pltpu.roll(m2, s, 1)
        m2 = jnp.maximum(jnp.minimum(m1, m1p), jnp.maximum(m2, m2p))
        m1 = jnp.maximum(m1, m1p)
    gh = mm_h(jnp.concatenate([m1 + m2, scores], axis=1), sgh_ref)
    gs = gh[:, 0:_N_GROUP]

    iota64 = jax.lax.broadcasted_iota(jnp.int32, (1, 64), 1)
    tie_pat = (iota64 // 8) < (iota64 % 8)

    def asc_rank8(v):
        a = jnp.concatenate([v] * 8, axis=1)
        b = mm_h(v, r8_ref)
        beats = (b < a) | ((b == a) & tie_pat)
        return mm_d(jnp.where(beats, 1.0, 0.0), m64_ref)

    drop = asc_rank8(gs) < (_N_DROP_GROUP - 0.5)

    heads = jnp.where(drop, gh[:, _N_GROUP:], 0.0)

    hh_rank = asc_rank8(heads)
    zbelow = jnp.where(heads > 0.0, 8.0,
                       jnp.where(heads == 0.0, kz_ref[...], 0.0))
    head_rank = hh_rank + zbelow

    iota8 = jax.lax.broadcasted_iota(jnp.int32, (1, 8), 1)
    nneg = mm_d(jnp.where(heads < 0.0, 1.0, 0.0), ones8_ref)
    e0 = jnp.where(heads[:, 0:1] == 0.0, 1.0, 0.0)
    e1 = jnp.where((heads[:, 1:2] == 0.0) & (iota8 == 7), 1.0, 0.0)
    zero_rank = nneg + e0 + e1 + iota8.astype(f32)

    crank = jnp.concatenate([head_rank, zero_rank], axis=1)
    orig_cand = mm_h(orig, scand_ref)

    crank_rep = jnp.concatenate([crank] * 8, axis=1)
    orig_rep = jnp.concatenate([orig_cand] * 8, axis=1)
    iota128 = jax.lax.broadcasted_iota(jnp.int32, (1, 128), 1)
    seg = (iota128 // 16).astype(f32)
    hit = crank_rep == seg
    inds = mm_d(jnp.where(hit, 1.0, 0.0), bdids_ref).astype(jnp.int32)
    valsden = mm_h(jnp.where(hit, orig_rep, 0.0), bdones_ref)
    denom = valsden[:, 8:9]
    sel = valsden[:, 0:8] / (denom + 1e-20) * _SCALE

    inds_ref[...] = inds
    sel_ref[...] = sel


def kernel(x, weight, e_score_correction_bias, interpret=False):
    n_tokens, d_model = x.shape
    n_experts = weight.shape[0]
    bias2d = e_score_correction_bias.reshape(1, n_experts)
    grid = (n_tokens // _BLOCK,)
    consts = tuple(jnp.asarray(c) for c in _CONSTS)
    const_specs = [
        pl.BlockSpec(c.shape, lambda i: (0, 0)) for c in consts]
    inds, sel = pl.pallas_call(
        _router_body,
        grid=grid,
        in_specs=[
            pl.BlockSpec((_BLOCK, d_model), lambda i: (i, 0)),
            pl.BlockSpec((n_experts, d_model), lambda i: (0, 0)),
            pl.BlockSpec((1, n_experts), lambda i: (0, 0)),
        ] + const_specs,
        out_specs=[
            pl.BlockSpec((_BLOCK, _TOP_K), lambda i: (i, 0)),
            pl.BlockSpec((_BLOCK, _TOP_K), lambda i: (i, 0)),
        ],
        out_shape=[
            jax.ShapeDtypeStruct((n_tokens, _TOP_K), jnp.int32),
            jax.ShapeDtypeStruct((n_tokens, _TOP_K), jnp.float32),
        ],
        compiler_params=pltpu.CompilerParams(
            dimension_semantics=("parallel",)),
        interpret=interpret,
    )(x, weight, bias2d, *consts)
    return inds, sel


# speedup vs baseline: 2.6842x; 1.0356x over previous
import numpy as np

import jax
import jax.numpy as jnp
from jax.experimental import pallas as pl
from jax.experimental.pallas import tpu as pltpu

_N_EXPERTS = 64
_TOP_K = 8
_N_GROUP = 8
_GROUP_SIZE = _N_EXPERTS // _N_GROUP
_N_DROP_GROUP = 4
_SCALE = 2.5
_BLOCK = 1024

_HEAD_IDS = [g * _GROUP_SIZE for g in range(_N_GROUP)]
_ZERO_IDS = [1, 2, 3, 4, 5, 6, 7, 9]
_CAND_IDS = np.array(_HEAD_IDS + _ZERO_IDS, dtype=np.float32)

_S_HEAD = np.zeros((_N_EXPERTS, _N_GROUP), np.float32)
for _g in range(_N_GROUP):
    _S_HEAD[_g * _GROUP_SIZE, _g] = 1.0
_S_BIG = np.zeros((2 * _N_EXPERTS, 264), np.float32)
for _i in range(64):
    _S_BIG[(_i // 8) * 8 + 7, _i] = 1.0
    _S_BIG[(_i % 8) * 8 + 7, 64 + _i] = 1.0
    _S_BIG[_N_EXPERTS + (_i // 8) * 8, 128 + _i] = 1.0
    _S_BIG[_N_EXPERTS + (_i % 8) * 8, 192 + _i] = 1.0
for _g in range(_N_GROUP):
    _S_BIG[_N_EXPERTS + 8 * _g, 256 + _g] = 1.0
_R8A = np.zeros((8, 128), np.float32)
for _i in range(64):
    _R8A[_i // 8, _i] = 1.0
    _R8A[_i % 8, 64 + _i] = 1.0
_S_CAND = np.zeros((_N_EXPERTS, 16), np.float32)
for _c, _e in enumerate(np.array(_HEAD_IDS + _ZERO_IDS)):
    _S_CAND[_e, _c] = 1.0
_R8 = np.zeros((8, 64), np.float32)
_M64 = np.zeros((64, 8), np.float32)
for _i in range(64):
    _R8[_i // 8, _i] = 1.0
    _M64[_i, _i % 8] = 1.0
_BD_IDS = np.zeros((128, 8), np.float32)
_BD_ONES9 = np.zeros((128, 9), np.float32)
for _r in range(8):
    for _c in range(16):
        _BD_IDS[_r * 16 + _c, _r] = float(_CAND_IDS[_c])
        _BD_ONES9[_r * 16 + _c, _r] = 1.0
_BD_ONES9[:, 8] = 1.0
_KZ = np.array([sum(1 for z in _ZERO_IDS if z < h) for h in _HEAD_IDS],
               np.float32).reshape(1, 8)

_ONES8 = np.ones((8, 8), np.float32)

_CONSTS = (_S_BIG, _S_CAND, _R8A, _M64, _BD_IDS, _BD_ONES9, _KZ, _ONES8)


def _router_body(x_ref, w_ref, b_ref, sbig_ref, scand_ref, r8a_ref,
                 m64_ref, bdids_ref, bdones_ref, kz_ref, ones8_ref,
                 inds_ref, sel_ref):
    blk = x_ref.shape[0]
    f32 = jnp.float32
    gates = jax.lax.dot_general(
        x_ref[...], w_ref[...], (((1,), (1,)), ((), ())),
        preferred_element_type=f32,
    )
    orig = jax.nn.sigmoid(gates)
    scores = orig + b_ref[...]

    def mm_h(a, m_ref):
        return jax.lax.dot_general(
            a, m_ref[...], (((1,), (0,)), ((), ())),
            preferred_element_type=f32,
            precision=jax.lax.Precision.HIGHEST)

    def mm_d(a, m_ref):
        return jax.lax.dot_general(
            a, m_ref[...], (((1,), (0,)), ((), ())),
            preferred_element_type=f32)

    m1p = pltpu.roll(scores, 1, 1)
    m1 = jnp.maximum(scores, m1p)
    m2 = jnp.minimum(scores, m1p)
    for s in (2, 4):
        m1p = pltpu.roll(m1, s, 1)
        m2p = pltpu.roll(m2, s, 1)
        m2 = jnp.maximum(jnp.minimum(m1, m1p), jnp.maximum(m2, m2p))
        m1 = jnp.maximum(m1, m1p)
    big = mm_h(jnp.concatenate([m1 + m2, scores], axis=1), sbig_ref)
    b_gs, a_gs = big[:, 0:64], big[:, 64:128]
    b_vr, a_vr = big[:, 128:192], big[:, 192:256]
    heads_raw = big[:, 256:264]

    iota64 = jax.lax.broadcasted_iota(jnp.int32, (1, 64), 1)
    tie_pat = (iota64 // 8) < (iota64 % 8)

    def count_beats(b, a):
        beats = (b < a) | ((b == a) & tie_pat)
        return mm_d(jnp.where(beats, 1.0, 0.0), m64_ref)

    drop = count_beats(b_gs, a_gs) < (_N_DROP_GROUP - 0.5)

    heads = jnp.where(drop, heads_raw, 0.0)

    dfe = mm_d(jnp.where(drop, 1.0, 0.0), r8a_ref)
    bm = jnp.where(dfe[:, 0:64] > 0.5, b_vr, 0.0)
    am = jnp.where(dfe[:, 64:128] > 0.5, a_vr, 0.0)
    hh_rank = count_beats(bm, am)
    zbelow = jnp.where(heads > 0.0, 8.0,
                       jnp.where(heads == 0.0, kz_ref[...], 0.0))
    head_rank = hh_rank + zbelow

    iota8 = jax.lax.broadcasted_iota(jnp.int32, (1, 8), 1)
    nneg = mm_d(jnp.where(heads < 0.0, 1.0, 0.0), ones8_ref)
    e0 = jnp.where(heads[:, 0:1] == 0.0, 1.0, 0.0)
    e1 = jnp.where((heads[:, 1:2] == 0.0) & (iota8 == 7), 1.0, 0.0)
    zero_rank = nneg + e0 + e1 + iota8.astype(f32)

    crank = jnp.concatenate([head_rank, zero_rank], axis=1)
    orig_cand = mm_h(orig, scand_ref)

    crank_rep = jnp.concatenate([crank] * 8, axis=1)
    orig_rep = jnp.concatenate([orig_cand] * 8, axis=1)
    iota128 = jax.lax.broadcasted_iota(jnp.int32, (1, 128), 1)
    seg = (iota128 // 16).astype(f32)
    hit = crank_rep == seg
    inds = mm_d(jnp.where(hit, 1.0, 0.0), bdids_ref).astype(jnp.int32)
    valsden = mm_h(jnp.where(hit, orig_rep, 0.0), bdones_ref)
    denom = valsden[:, 8:9]
    sel = valsden[:, 0:8] / (denom + 1e-20) * _SCALE

    inds_ref[...] = inds
    sel_ref[...] = sel


def kernel(x, weight, e_score_correction_bias, interpret=False):
    n_tokens, d_model = x.shape
    n_experts = weight.shape[0]
    bias2d = e_score_correction_bias.reshape(1, n_experts)
    grid = (n_tokens // _BLOCK,)
    consts = tuple(jnp.asarray(c) for c in _CONSTS)
    const_specs = [
        pl.BlockSpec(c.shape, lambda i: (0, 0)) for c in consts]
    inds, sel = pl.pallas_call(
        _router_body,
        grid=grid,
        in_specs=[
            pl.BlockSpec((_BLOCK, d_model), lambda i: (i, 0)),
            pl.BlockSpec((n_experts, d_model), lambda i: (0, 0)),
            pl.BlockSpec((1, n_experts), lambda i: (0, 0)),
        ] + const_specs,
        out_specs=[
            pl.BlockSpec((_BLOCK, _TOP_K), lambda i: (i, 0)),
            pl.BlockSpec((_BLOCK, _TOP_K), lambda i: (i, 0)),
        ],
        out_shape=[
            jax.ShapeDtypeStruct((n_tokens, _TOP_K), jnp.int32),
            jax.ShapeDtypeStruct((n_tokens, _TOP_K), jnp.float32),
        ],
        compiler_params=pltpu.CompilerParams(
            dimension_semantics=("parallel",)),
        interpret=interpret,
    )(x, weight, bias2d, *consts)
    return inds, sel
